# baseline (speedup 1.0000x reference)
import functools

import jax
import jax.numpy as jnp
from jax import lax
from jax.experimental import pallas as pl
from jax.experimental.pallas import tpu as pltpu
from jax.experimental.pallas import tpu_sc as plsc

NF = 16
CHUNK = 128


@functools.lru_cache(maxsize=None)
def _build(B, NC, NS):
    NW = NC * NS
    b_per_w = B // NW
    n_chunks = b_per_w // CHUNK
    mesh = plsc.VectorSubcoreMesh(core_axis_name="c", subcore_axis_name="s")

    @functools.partial(
        pl.kernel,
        mesh=mesh,
        out_type=jax.ShapeDtypeStruct((B,), jnp.float32),
        compiler_params=pltpu.CompilerParams(
            needs_layout_passes=False, use_tc_tiling_on_sc=False),
        scratch_types=[
            pltpu.VMEM((n_chunks, CHUNK), jnp.int32),
            pltpu.VMEM((n_chunks, CHUNK), jnp.int32),
            pltpu.VMEM((b_per_w, NF), jnp.float32),
            pltpu.VMEM((b_per_w, NF), jnp.float32),
            pltpu.VMEM((32,), jnp.float32),
            pltpu.VMEM((b_per_w,), jnp.float32),
            pltpu.SemaphoreType.DMA,
        ],
    )
    def k(ut_hbm, it_hbm, u3_hbm, i3_hbm, wb_hbm, out_hbm,
          uidx_v, iidx_v, urows_v, irows_v, wb_v, out_v, sem):
        wid = lax.axis_index("s") * NC + lax.axis_index("c")
        base = wid * b_per_w
        pltpu.sync_copy(u3_hbm.at[wid], uidx_v)
        pltpu.sync_copy(i3_hbm.at[wid], iidx_v)
        pltpu.sync_copy(wb_hbm, wb_v)
        copies = []
        for c in range(n_chunks):
            copies.append(pltpu.async_copy(
                ut_hbm.at[uidx_v.at[c]],
                urows_v.at[pl.ds(c * CHUNK, CHUNK), :], sem))
            copies.append(pltpu.async_copy(
                it_hbm.at[iidx_v.at[c]],
                irows_v.at[pl.ds(c * CHUNK, CHUNK), :], sem))
        for cp in copies:
            cp.wait()

        wv = wb_v[pl.ds(0, 16)]
        bv = wb_v[pl.ds(16, 16)]
        w_s = [wv[f] for f in range(NF)]
        bias = bv[0]
        lane = jnp.arange(16, dtype=jnp.int32)

        def g_body(g, carry):
            rows = g * 16 + lane
            acc = jnp.full((16,), bias, jnp.float32)
            for f in range(NF):
                cf = jnp.full((16,), f, jnp.int32)
                uu = plsc.load_gather(urows_v, [rows, cf])
                vv = plsc.load_gather(irows_v, [rows, cf])
                acc = acc + uu * vv * w_s[f]
            out_v[pl.ds(g * 16, 16)] = acc
            return carry

        lax.fori_loop(0, b_per_w // 16, g_body, 0)
        pltpu.sync_copy(out_v, out_hbm.at[pl.ds(base, b_per_w)])

    return k


def kernel(U_ids, I_ids, user_table, item_table, W, b):
    B = U_ids.shape[0]
    info = plsc.get_sparse_core_info()
    NC, NS = info.num_cores, info.num_subcores
    NW = NC * NS
    n_chunks = (B // NW) // CHUNK
    u3 = U_ids.astype(jnp.int32).reshape(NW, n_chunks, CHUNK)
    i3 = I_ids.astype(jnp.int32).reshape(NW, n_chunks, CHUNK)
    wb = jnp.zeros((32,), jnp.float32).at[:NF].set(W.reshape(-1)).at[NF].set(b[0])
    return _build(B, NC, NS)(user_table, item_table, u3, i3, wb)


# speedup vs baseline: 3.7854x; 3.7854x over previous
import functools

import jax
import jax.numpy as jnp
from jax import lax
from jax.experimental import pallas as pl
from jax.experimental.pallas import tpu as pltpu
from jax.experimental.pallas import tpu_sc as plsc

NF = 16
LANE = 128


@functools.lru_cache(maxsize=None)
def _build(B, N, NC, NS):
    NW = NC * NS
    b_per_w = B // NW
    n_groups = b_per_w // 16
    n_tc = N // LANE
    tail0 = n_tc * LANE
    n_tail = N - tail0
    mesh = plsc.VectorSubcoreMesh(core_axis_name="c", subcore_axis_name="s")

    @functools.partial(
        pl.kernel,
        mesh=mesh,
        out_type=jax.ShapeDtypeStruct((B,), jnp.float32),
        compiler_params=pltpu.CompilerParams(needs_layout_passes=False),
        scratch_types=[
            pltpu.VMEM((1, b_per_w), jnp.int32),
            pltpu.VMEM((1, b_per_w), jnp.int32),
            pltpu.VMEM((16, NF, LANE), jnp.float32),
            pltpu.VMEM((16, NF, LANE), jnp.float32),
            pltpu.VMEM((NF, n_tail), jnp.float32),
            pltpu.VMEM((NF, n_tail), jnp.float32),
            pltpu.VMEM((32,), jnp.float32),
            pltpu.VMEM((b_per_w,), jnp.float32),
            pltpu.SemaphoreType.DMA,
        ],
    )
    def k(ut_hbm, it_hbm, u2_hbm, i2_hbm, ut_tail_hbm, it_tail_hbm, wb_hbm,
          out_hbm, uid_v, iid_v, uslab_v, islab_v, utail_v, itail_v, wb_v,
          out_v, sem):
        wid = lax.axis_index("s") * NC + lax.axis_index("c")
        base = wid * b_per_w
        pltpu.sync_copy(u2_hbm.at[wid], uid_v)
        pltpu.sync_copy(i2_hbm.at[wid], iid_v)
        pltpu.sync_copy(ut_tail_hbm, utail_v)
        pltpu.sync_copy(it_tail_hbm, itail_v)
        pltpu.sync_copy(wb_hbm, wb_v)

        wv = wb_v[pl.ds(0, 16)]
        bv = wb_v[pl.ds(16, 16)]
        bias = bv[0]
        lane16 = jnp.arange(16, dtype=jnp.int32)

        def g_body(g, carry):
            o = g * 16
            uu = uid_v[0, pl.ds(o, 16)]
            vv = iid_v[0, pl.ds(o, 16)]
            acc = jnp.full((16,), 0.0, jnp.float32)
            for sub in range(4):
                copies = []
                for j in range(sub * 4, sub * 4 + 4):
                    utc = jnp.minimum(uu[j] // LANE, n_tc - 1)
                    itc = jnp.minimum(vv[j] // LANE, n_tc - 1)
                    copies.append(pltpu.async_copy(
                        ut_hbm.at[:, pl.ds(pl.multiple_of(utc * LANE, LANE), LANE)],
                        uslab_v.at[j], sem))
                    copies.append(pltpu.async_copy(
                        it_hbm.at[:, pl.ds(pl.multiple_of(itc * LANE, LANE), LANE)],
                        islab_v.at[j], sem))
                for cp in copies:
                    cp.wait()
                for j in range(sub * 4, sub * 4 + 4):
                    jv = jnp.full((16,), j, jnp.int32)
                    uid = uu[j]
                    iid = vv[j]
                    ucol = plsc.load_gather(
                        uslab_v,
                        [jv, lane16, jnp.full((16,), uid % LANE, jnp.int32)])
                    icol = plsc.load_gather(
                        islab_v,
                        [jv, lane16, jnp.full((16,), iid % LANE, jnp.int32)])
                    ut_col = plsc.load_gather(
                        utail_v,
                        [lane16, jnp.full(
                            (16,),
                            jnp.clip(uid - tail0, 0, n_tail - 1), jnp.int32)])
                    it_col = plsc.load_gather(
                        itail_v,
                        [lane16, jnp.full(
                            (16,),
                            jnp.clip(iid - tail0, 0, n_tail - 1), jnp.int32)])
                    ucol = jnp.where(uid >= tail0, ut_col, ucol)
                    icol = jnp.where(iid >= tail0, it_col, icol)
                    s = lax.reduce_sum_p.bind(ucol * icol * wv, axes=(0,))
                    acc = jnp.where(lane16 == j, s, acc)
            out_v[pl.ds(o, 16)] = acc + bias
            return carry

        lax.fori_loop(0, n_groups, g_body, 0)
        pltpu.sync_copy(out_v, out_hbm.at[pl.ds(base, b_per_w)])

    return k


def kernel(U_ids, I_ids, user_table, item_table, W, b):
    B = U_ids.shape[0]
    N = user_table.shape[0]
    info = plsc.get_sparse_core_info()
    NC, NS = info.num_cores, info.num_subcores
    NW = NC * NS
    u2 = U_ids.astype(jnp.int32).reshape(NW, 1, B // NW)
    i2 = I_ids.astype(jnp.int32).reshape(NW, 1, B // NW)
    wb = jnp.zeros((32,), jnp.float32).at[:NF].set(W.reshape(-1)).at[NF].set(b[0])
    tail0 = (N // LANE) * LANE
    ut_tail = user_table[tail0:, :].T
    it_tail = item_table[tail0:, :].T
    return _build(B, N, NC, NS)(
        user_table.T, item_table.T, u2, i2, ut_tail, it_tail, wb)


# speedup vs baseline: 4.0008x; 1.0569x over previous
import functools

import jax
import jax.numpy as jnp
from jax import lax
from jax.experimental import pallas as pl
from jax.experimental.pallas import tpu as pltpu
from jax.experimental.pallas import tpu_sc as plsc

NF = 16
LANE = 128


@functools.lru_cache(maxsize=None)
def _build(B, N, NC, NS):
    NW = NC * NS
    b_per_w = B // NW
    n_groups = b_per_w // 16
    n_tc = N // LANE
    tail0 = n_tc * LANE
    n_tail = N - tail0
    mesh = plsc.VectorSubcoreMesh(core_axis_name="c", subcore_axis_name="s")

    @functools.partial(
        pl.kernel,
        mesh=mesh,
        out_type=jax.ShapeDtypeStruct((B,), jnp.float32),
        compiler_params=pltpu.CompilerParams(needs_layout_passes=False),
        scratch_types=[
            pltpu.VMEM((1, b_per_w), jnp.int32),
            pltpu.VMEM((1, b_per_w), jnp.int32),
            pltpu.VMEM((16, NF, LANE), jnp.float32),
            pltpu.VMEM((16, NF, LANE), jnp.float32),
            pltpu.VMEM((NF, n_tail), jnp.float32),
            pltpu.VMEM((NF, n_tail), jnp.float32),
            pltpu.VMEM((32,), jnp.float32),
            pltpu.VMEM((b_per_w,), jnp.float32),
            pltpu.SemaphoreType.DMA,
        ],
    )
    def k(ut_hbm, it_hbm, u2_hbm, i2_hbm, ut_tail_hbm, it_tail_hbm, wb_hbm,
          out_hbm, uid_v, iid_v, uslab_v, islab_v, utail_v, itail_v, wb_v,
          out_v, sem):
        wid = lax.axis_index("s") * NC + lax.axis_index("c")
        base = wid * b_per_w
        pltpu.sync_copy(u2_hbm.at[wid], uid_v)
        pltpu.sync_copy(i2_hbm.at[wid], iid_v)
        pltpu.sync_copy(ut_tail_hbm, utail_v)
        pltpu.sync_copy(it_tail_hbm, itail_v)
        pltpu.sync_copy(wb_hbm, wb_v)

        wv = wb_v[pl.ds(0, 16)]
        bv = wb_v[pl.ds(16, 16)]
        w_s = [wv[f] for f in range(NF)]
        bias = bv[0]
        lane16 = jnp.arange(16, dtype=jnp.int32)

        def g_body(g, carry):
            o = g * 16
            uu = uid_v[0, pl.ds(o, 16)]
            vv = iid_v[0, pl.ds(o, 16)]
            for sub in range(4):
                copies = []
                for j in range(sub * 4, sub * 4 + 4):
                    utc = jnp.minimum(uu[j] // LANE, n_tc - 1)
                    itc = jnp.minimum(vv[j] // LANE, n_tc - 1)
                    copies.append(pltpu.async_copy(
                        ut_hbm.at[:, pl.ds(pl.multiple_of(utc * LANE, LANE), LANE)],
                        uslab_v.at[j], sem))
                    copies.append(pltpu.async_copy(
                        it_hbm.at[:, pl.ds(pl.multiple_of(itc * LANE, LANE), LANE)],
                        islab_v.at[j], sem))
                for cp in copies:
                    cp.wait()
            uoff = uu % LANE
            ioff = vv % LANE
            umask = uu >= tail0
            imask = vv >= tail0
            uto = jnp.clip(uu - tail0, 0, n_tail - 1)
            ito = jnp.clip(vv - tail0, 0, n_tail - 1)
            acc = jnp.full((16,), 0.0, jnp.float32)
            for f in range(NF):
                fv = jnp.full((16,), f, jnp.int32)
                u_f = plsc.load_gather(uslab_v, [lane16, fv, uoff])
                u_f = jnp.where(umask, plsc.load_gather(utail_v, [fv, uto]), u_f)
                i_f = plsc.load_gather(islab_v, [lane16, fv, ioff])
                i_f = jnp.where(imask, plsc.load_gather(itail_v, [fv, ito]), i_f)
                acc = acc + u_f * i_f * w_s[f]
            out_v[pl.ds(o, 16)] = acc + bias
            return carry

        lax.fori_loop(0, n_groups, g_body, 0)
        pltpu.sync_copy(out_v, out_hbm.at[pl.ds(base, b_per_w)])

    return k


def kernel(U_ids, I_ids, user_table, item_table, W, b):
    B = U_ids.shape[0]
    N = user_table.shape[0]
    info = plsc.get_sparse_core_info()
    NC, NS = info.num_cores, info.num_subcores
    NW = NC * NS
    u2 = U_ids.astype(jnp.int32).reshape(NW, 1, B // NW)
    i2 = I_ids.astype(jnp.int32).reshape(NW, 1, B // NW)
    wb = jnp.zeros((32,), jnp.float32).at[:NF].set(W.reshape(-1)).at[NF].set(b[0])
    tail0 = (N // LANE) * LANE
    ut_tail = user_table[tail0:, :].T
    it_tail = item_table[tail0:, :].T
    return _build(B, N, NC, NS)(
        user_table.T, item_table.T, u2, i2, ut_tail, it_tail, wb)


# speedup vs baseline: 5.8226x; 1.4553x over previous
import functools

import jax
import jax.numpy as jnp
from jax import lax
from jax.experimental import pallas as pl
from jax.experimental.pallas import tpu as pltpu
from jax.experimental.pallas import tpu_sc as plsc

NF = 16
LANE = 128


@functools.lru_cache(maxsize=None)
def _build(B, N, NC, NS):
    NW = NC * NS
    b_per_w = B // NW
    n_groups = b_per_w // 16
    n_tc = N // LANE
    tail0 = n_tc * LANE
    n_tail = N - tail0
    mesh = plsc.VectorSubcoreMesh(core_axis_name="c", subcore_axis_name="s")

    @functools.partial(
        pl.kernel,
        mesh=mesh,
        out_type=jax.ShapeDtypeStruct((B,), jnp.float32),
        compiler_params=pltpu.CompilerParams(needs_layout_passes=False),
        scratch_types=[
            pltpu.VMEM((1, b_per_w), jnp.int32),
            pltpu.VMEM((1, b_per_w), jnp.int32),
            pltpu.VMEM((16, NF, LANE), jnp.float32),
            pltpu.VMEM((16, NF, LANE), jnp.float32),
            pltpu.VMEM((NF, n_tail), jnp.float32),
            pltpu.VMEM((NF, n_tail), jnp.float32),
            pltpu.VMEM((32,), jnp.float32),
            pltpu.VMEM((b_per_w,), jnp.float32),
            pltpu.SemaphoreType.DMA,
            pltpu.SemaphoreType.DMA,
        ],
    )
    def k(ut_hbm, it_hbm, u2_hbm, i2_hbm, ut_tail_hbm, it_tail_hbm, wb_hbm,
          out_hbm, uid_v, iid_v, uslab_v, islab_v, utail_v, itail_v, wb_v,
          out_v, sem_a, sem_b):
        wid = lax.axis_index("s") * NC + lax.axis_index("c")
        base = wid * b_per_w
        pltpu.sync_copy(u2_hbm.at[wid], uid_v)
        pltpu.sync_copy(i2_hbm.at[wid], iid_v)
        pltpu.sync_copy(ut_tail_hbm, utail_v)
        pltpu.sync_copy(it_tail_hbm, itail_v)
        pltpu.sync_copy(wb_hbm, wb_v)

        wv = wb_v[pl.ds(0, 16)]
        bv = wb_v[pl.ds(16, 16)]
        w_s = [wv[f] for f in range(NF)]
        bias = bv[0]
        lane16 = jnp.arange(16, dtype=jnp.int32)

        def fire(uu, vv, lo, sem):
            for j in range(lo, lo + 8):
                utc = jnp.minimum(uu[j] // LANE, n_tc - 1)
                itc = jnp.minimum(vv[j] // LANE, n_tc - 1)
                pltpu.async_copy(
                    ut_hbm.at[:, pl.ds(pl.multiple_of(utc * LANE, LANE), LANE)],
                    uslab_v.at[j], sem)
                pltpu.async_copy(
                    it_hbm.at[:, pl.ds(pl.multiple_of(itc * LANE, LANE), LANE)],
                    islab_v.at[j], sem)

        def drain(lo, sem):
            for j in range(lo, lo + 8):
                pltpu.make_async_copy(
                    ut_hbm.at[:, pl.ds(0, LANE)], uslab_v.at[j], sem).wait()
                pltpu.make_async_copy(
                    it_hbm.at[:, pl.ds(0, LANE)], islab_v.at[j], sem).wait()

        def extract(uu, vv):
            uoff = uu % LANE
            ioff = vv % LANE
            umask = uu >= tail0
            imask = vv >= tail0
            uto = jnp.clip(uu - tail0, 0, n_tail - 1)
            ito = jnp.clip(vv - tail0, 0, n_tail - 1)
            acc = jnp.full((16,), 0.0, jnp.float32)
            for f in range(NF):
                fv = jnp.full((16,), f, jnp.int32)
                u_f = plsc.load_gather(uslab_v, [lane16, fv, uoff])
                u_f = jnp.where(umask, plsc.load_gather(utail_v, [fv, uto]), u_f)
                i_f = plsc.load_gather(islab_v, [lane16, fv, ioff])
                i_f = jnp.where(imask, plsc.load_gather(itail_v, [fv, ito]), i_f)
                acc = acc + u_f * i_f * w_s[f]
            return acc

        half0 = lane16 < 8

        uu0 = uid_v[0, pl.ds(0, 16)]
        vv0 = iid_v[0, pl.ds(0, 16)]
        fire(uu0, vv0, 0, sem_a)

        def g_body(g, carry):
            o = g * 16
            uu = uid_v[0, pl.ds(o, 16)]
            vv = iid_v[0, pl.ds(o, 16)]
            on = jnp.minimum(g + 1, n_groups - 1) * 16
            uun = uid_v[0, pl.ds(on, 16)]
            vvn = iid_v[0, pl.ds(on, 16)]
            fire(uu, vv, 8, sem_b)
            drain(0, sem_a)
            acc_a = extract(uu, vv)
            fire(uun, vvn, 0, sem_a)
            drain(8, sem_b)
            acc_b = extract(uu, vv)
            out_v[pl.ds(o, 16)] = jnp.where(half0, acc_a, acc_b) + bias
            return carry

        lax.fori_loop(0, n_groups, g_body, 0)
        drain(0, sem_a)
        pltpu.sync_copy(out_v, out_hbm.at[pl.ds(base, b_per_w)])

    return k


def kernel(U_ids, I_ids, user_table, item_table, W, b):
    B = U_ids.shape[0]
    N = user_table.shape[0]
    info = plsc.get_sparse_core_info()
    NC, NS = info.num_cores, info.num_subcores
    NW = NC * NS
    u2 = U_ids.astype(jnp.int32).reshape(NW, 1, B // NW)
    i2 = I_ids.astype(jnp.int32).reshape(NW, 1, B // NW)
    wb = jnp.zeros((32,), jnp.float32).at[:NF].set(W.reshape(-1)).at[NF].set(b[0])
    tail0 = (N // LANE) * LANE
    ut_tail = user_table[tail0:, :].T
    it_tail = item_table[tail0:, :].T
    return _build(B, N, NC, NS)(
        user_table.T, item_table.T, u2, i2, ut_tail, it_tail, wb)


# speedup vs baseline: 6.0952x; 1.0468x over previous
import functools

import jax
import jax.numpy as jnp
from jax import lax
from jax.experimental import pallas as pl
from jax.experimental.pallas import tpu as pltpu
from jax.experimental.pallas import tpu_sc as plsc

NF = 16
LANE = 128


@functools.lru_cache(maxsize=None)
def _build(B, N, NC, NS):
    NW = NC * NS
    b_per_w = B // NW
    n_groups = b_per_w // 16
    n_tc = N // LANE
    tail0 = n_tc * LANE
    n_tail = N - tail0
    mesh = plsc.VectorSubcoreMesh(core_axis_name="c", subcore_axis_name="s")

    @functools.partial(
        pl.kernel,
        mesh=mesh,
        out_type=jax.ShapeDtypeStruct((B,), jnp.float32),
        compiler_params=pltpu.CompilerParams(needs_layout_passes=False),
        scratch_types=[
            pltpu.VMEM((1, b_per_w), jnp.int32),
            pltpu.VMEM((1, b_per_w), jnp.int32),
            pltpu.VMEM((16, NF, LANE), jnp.float32),
            pltpu.VMEM((16, NF, LANE), jnp.float32),
            pltpu.VMEM((NF, n_tail), jnp.float32),
            pltpu.VMEM((NF, n_tail), jnp.float32),
            pltpu.VMEM((32,), jnp.float32),
            pltpu.VMEM((b_per_w,), jnp.float32),
            pltpu.SemaphoreType.DMA,
            pltpu.SemaphoreType.DMA,
        ],
    )
    def k(ut_hbm, it_hbm, u2_hbm, i2_hbm, ut_tail_hbm, it_tail_hbm, wb_hbm,
          out_hbm, uid_v, iid_v, uslab_v, islab_v, utail_v, itail_v, wb_v,
          out_v, sem_a, sem_b):
        wid = lax.axis_index("s") * NC + lax.axis_index("c")
        base = wid * b_per_w
        pltpu.sync_copy(u2_hbm.at[wid], uid_v)
        pltpu.sync_copy(i2_hbm.at[wid], iid_v)
        pltpu.sync_copy(ut_tail_hbm, utail_v)
        pltpu.sync_copy(it_tail_hbm, itail_v)
        pltpu.sync_copy(wb_hbm, wb_v)

        wv = wb_v[pl.ds(0, 16)]
        bv = wb_v[pl.ds(16, 16)]
        w_s = [wv[f] for f in range(NF)]
        bias = bv[0]
        lane16 = jnp.arange(16, dtype=jnp.int32)

        def extract(uu, vv):
            uoff = uu % LANE
            ioff = vv % LANE
            umask = uu >= tail0
            imask = vv >= tail0
            uto = jnp.clip(uu - tail0, 0, n_tail - 1)
            ito = jnp.clip(vv - tail0, 0, n_tail - 1)
            acc = jnp.full((16,), 0.0, jnp.float32)
            for f in range(NF):
                fv = jnp.full((16,), f, jnp.int32)
                u_f = plsc.load_gather(uslab_v, [lane16, fv, uoff])
                u_f = jnp.where(umask, plsc.load_gather(utail_v, [fv, uto]), u_f)
                i_f = plsc.load_gather(islab_v, [lane16, fv, ioff])
                i_f = jnp.where(imask, plsc.load_gather(itail_v, [fv, ito]), i_f)
                acc = acc + u_f * i_f * w_s[f]
            return acc

        def g_body(g, carry):
            o = g * 16
            uu = uid_v[0, pl.ds(o, 16)]
            vv = iid_v[0, pl.ds(o, 16)]
            copies = []
            for j in range(16):
                utc = jnp.minimum(uu[j] // LANE, n_tc - 1)
                itc = jnp.minimum(vv[j] // LANE, n_tc - 1)
                copies.append(pltpu.async_copy(
                    ut_hbm.at[:, pl.ds(pl.multiple_of(utc * LANE, LANE), LANE)],
                    uslab_v.at[j], sem_a))
                copies.append(pltpu.async_copy(
                    it_hbm.at[:, pl.ds(pl.multiple_of(itc * LANE, LANE), LANE)],
                    islab_v.at[j], sem_a))
            for cp in copies:
                cp.wait()
            out_v[pl.ds(o, 16)] = extract(uu, vv) + bias
            return carry

        lax.fori_loop(0, n_groups, g_body, 0)
        pltpu.sync_copy(out_v, out_hbm.at[pl.ds(base, b_per_w)])

    return k


def kernel(U_ids, I_ids, user_table, item_table, W, b):
    B = U_ids.shape[0]
    N = user_table.shape[0]
    info = plsc.get_sparse_core_info()
    NC, NS = info.num_cores, info.num_subcores
    NW = NC * NS
    u2 = U_ids.astype(jnp.int32).reshape(NW, 1, B // NW)
    i2 = I_ids.astype(jnp.int32).reshape(NW, 1, B // NW)
    wb = jnp.zeros((32,), jnp.float32).at[:NF].set(W.reshape(-1)).at[NF].set(b[0])
    tail0 = (N // LANE) * LANE
    ut_tail = user_table[tail0:, :].T
    it_tail = item_table[tail0:, :].T
    return _build(B, N, NC, NS)(
        user_table.T, item_table.T, u2, i2, ut_tail, it_tail, wb)
